# baseline (speedup 1.0000x reference)
import functools

import jax
import jax.numpy as jnp
from jax import lax
from jax.experimental import pallas as pl
from jax.experimental.pallas import tpu as pltpu
from jax.experimental.pallas import tpu_sc as plsc

_EMBED = 256
_HEADS = 8
_POINTS = 4
_NCOL = _HEADS * 4 * _POINTS


def _vproj_body(vm_ref, wv_ref, bv_ref, out_ref):
    v = jnp.dot(vm_ref[0], wv_ref[...],
                preferred_element_type=jnp.float32) + bv_ref[...]
    out_ref[0] = v[:, :128]
    out_ref[1] = v[:, 128:]


def _value_proj(value_map, W_val, b_val, T=2000):
    B, C, H, W = value_map.shape
    HW = H * W
    vm = jnp.transpose(value_map.reshape(B, C, HW), (0, 2, 1))
    nt = HW // T
    return pl.pallas_call(
        _vproj_body,
        grid=(B, nt),
        in_specs=[
            pl.BlockSpec((1, T, C), lambda b, t: (b, t, 0)),
            pl.BlockSpec((C, C), lambda b, t: (0, 0)),
            pl.BlockSpec((1, C), lambda b, t: (0, 0)),
        ],
        out_specs=pl.BlockSpec((2, T, 128), lambda b, t, n=nt: (0, b * n + t, 0)),
        out_shape=jax.ShapeDtypeStruct((2, B * HW, 128), jnp.float32),
    )(vm, W_val, b_val.reshape(1, C))


def _idxw_body(q_ref, rpx_ref, rpy_ref, wox_ref, woy_ref, box_ref, boy_ref,
               wa_ref, ba_ref, gsum_ref, idx_ref, w_ref, *, H, W, B):
    Tq = q_ref.shape[1]
    q = q_ref[0]
    offx = jnp.dot(q, wox_ref[...], preferred_element_type=jnp.float32) + box_ref[...]
    offy = jnp.dot(q, woy_ref[...], preferred_element_type=jnp.float32) + boy_ref[...]
    logits = jnp.dot(q, wa_ref[...], preferred_element_type=jnp.float32) + ba_ref[...]
    m = jnp.max(logits, axis=-1, keepdims=True)
    e = jnp.exp(logits - m)
    esum = jnp.dot(e, gsum_ref[...], preferred_element_type=jnp.float32)
    attn = e / esum

    x = (rpx_ref[0] + offx) * W - 0.5
    y = (rpy_ref[0] + offy) * H - 0.5
    x0 = jnp.floor(x)
    fx = x - x0
    y0 = jnp.floor(y)
    fy = y - y0

    col = lax.broadcasted_iota(jnp.int32, (Tq, _NCOL), 1)
    cvec = (col % 16) // 4
    is_x1 = (cvec % 2) == 1
    is_y1 = cvec >= 2

    one = jnp.float32(1.0)
    wx = jnp.where(is_x1, fx, one - fx)
    wy = jnp.where(is_y1, fy, one - fy)
    xf = jnp.where(is_x1, x0 + one, x0)
    yf = jnp.where(is_y1, y0 + one, y0)
    vx = ((xf >= 0.0) & (xf <= W - 1.0)).astype(jnp.float32)
    vy = ((yf >= 0.0) & (yf <= H - 1.0)).astype(jnp.float32)
    xc = jnp.clip(xf, 0.0, W - 1.0).astype(jnp.int32)
    yc = jnp.clip(yf, 0.0, H - 1.0).astype(jnp.int32)

    half = col // 64
    hh = (col // 16) % 4
    base = pl.program_id(0) * (H * W)
    idx_ref[0] = half * (B * H * W * 4) + (base + yc * W + xc) * 4 + hh
    w_ref[0] = wx * wy * vx * vy * attn


def _idx_weights(query, reference_points, W_off, b_off, W_attn, b_attn,
                 H, W, Tq=1000):
    B, Nq, C = query.shape
    Wo = W_off.reshape(C, _HEADS, _POINTS, 2)
    bo = b_off.reshape(_HEADS, _POINTS, 2)
    wox = jnp.broadcast_to(Wo[:, :, None, :, 0], (C, _HEADS, 4, _POINTS)).reshape(C, _NCOL)
    woy = jnp.broadcast_to(Wo[:, :, None, :, 1], (C, _HEADS, 4, _POINTS)).reshape(C, _NCOL)
    box = jnp.broadcast_to(bo[:, None, :, 0], (_HEADS, 4, _POINTS)).reshape(1, _NCOL)
    boy = jnp.broadcast_to(bo[:, None, :, 1], (_HEADS, 4, _POINTS)).reshape(1, _NCOL)
    Wa = W_attn.reshape(C, _HEADS, _POINTS)
    wa = jnp.broadcast_to(Wa[:, :, None, :], (C, _HEADS, 4, _POINTS)).reshape(C, _NCOL)
    ba = jnp.broadcast_to(b_attn.reshape(_HEADS, _POINTS)[:, None, :],
                          (_HEADS, 4, _POINTS)).reshape(1, _NCOL)
    gsum = jnp.kron(jnp.eye(_NCOL // 4, dtype=jnp.float32),
                    jnp.ones((4, 4), jnp.float32))
    rpx = reference_points[:, :, 0:1]
    rpy = reference_points[:, :, 1:2]

    body = functools.partial(_idxw_body, H=H, W=W, B=B)
    return pl.pallas_call(
        body,
        grid=(B, Nq // Tq),
        in_specs=[
            pl.BlockSpec((1, Tq, C), lambda b, t: (b, t, 0)),
            pl.BlockSpec((1, Tq, 1), lambda b, t: (b, t, 0)),
            pl.BlockSpec((1, Tq, 1), lambda b, t: (b, t, 0)),
            pl.BlockSpec((C, _NCOL), lambda b, t: (0, 0)),
            pl.BlockSpec((C, _NCOL), lambda b, t: (0, 0)),
            pl.BlockSpec((1, _NCOL), lambda b, t: (0, 0)),
            pl.BlockSpec((1, _NCOL), lambda b, t: (0, 0)),
            pl.BlockSpec((C, _NCOL), lambda b, t: (0, 0)),
            pl.BlockSpec((1, _NCOL), lambda b, t: (0, 0)),
            pl.BlockSpec((_NCOL, _NCOL), lambda b, t: (0, 0)),
        ],
        out_specs=[
            pl.BlockSpec((1, Tq, _NCOL), lambda b, t: (b, t, 0)),
            pl.BlockSpec((1, Tq, _NCOL), lambda b, t: (b, t, 0)),
        ],
        out_shape=[
            jax.ShapeDtypeStruct((B, Nq, _NCOL), jnp.int32),
            jax.ShapeDtypeStruct((B, Nq, _NCOL), jnp.float32),
        ],
    )(query, rpx, rpy, wox, woy, box, boy, wa, ba, gsum)


_GW = 128


def _sc_gather(table, idx_flat):
    S4 = idx_flat.shape[0]
    D = table.shape[1]
    mesh = plsc.VectorSubcoreMesh(core_axis_name="core",
                                  subcore_axis_name="subcore")

    @functools.partial(
        pl.kernel,
        out_type=jax.ShapeDtypeStruct((S4, D), table.dtype),
        mesh=mesh,
        compiler_params=pltpu.CompilerParams(use_tc_tiling_on_sc=False),
    )
    def gather_kernel(table_hbm, idx_hbm, out_hbm):
        def body(i_vmem, o_vmem):
            pltpu.sync_copy(table_hbm.at[i_vmem.at[0]], o_vmem)

        pltpu.emit_pipeline(
            body,
            grid=(S4 // _GW,),
            in_specs=[pl.BlockSpec((1, _GW), index_map=lambda i: (0, i))],
            out_specs=[pl.BlockSpec((_GW, D), index_map=lambda i: (i, 0))],
            core_axis_name=("core", "subcore"),
            dimension_semantics=(pltpu.PARALLEL,),
        )(idx_hbm, out_hbm)

    return gather_kernel(table, idx_flat.reshape(1, S4))


def _combine_body(g_ref, we_ref, rw_ref, bo_ref, out_ref, *, Tq):
    g = g_ref[...]
    we = we_ref[...]
    gw = (g * we.astype(jnp.float32)).astype(jnp.bfloat16)
    gw3 = gw.reshape(Tq, 32, 128)
    res = lax.dot_general(gw3, rw_ref[...],
                          (((2,), (1,)), ((1,), (0,))),
                          preferred_element_type=jnp.float32)
    out_ref[0] = res.sum(axis=0) + bo_ref[...]


def _combine(gathered, w, W_out, b_out, B, Nq, Tq=400):
    g = gathered.reshape(B * Nq * 32, 128)
    wexp = jnp.broadcast_to(
        w.reshape(B * Nq, 32, 4, 1).astype(jnp.bfloat16),
        (B * Nq, 32, 4, 32)).reshape(B * Nq * 32, 128)
    rw3 = jnp.broadcast_to(
        W_out.reshape(_HEADS, 1, 1, 32, _EMBED),
        (_HEADS, 4, 4, 32, _EMBED)).reshape(32, 128, _EMBED).astype(jnp.bfloat16)
    body = functools.partial(_combine_body, Tq=Tq)
    return pl.pallas_call(
        body,
        grid=(B * Nq // Tq,),
        in_specs=[
            pl.BlockSpec((Tq * 32, 128), lambda t: (t, 0)),
            pl.BlockSpec((Tq * 32, 128), lambda t: (t, 0)),
            pl.BlockSpec((32, 128, _EMBED), lambda t: (0, 0, 0)),
            pl.BlockSpec((1, _EMBED), lambda t: (0, 0)),
        ],
        out_specs=pl.BlockSpec((1, Tq, _EMBED), lambda t: (0, t, 0)),
        out_shape=jax.ShapeDtypeStruct((1, B * Nq, _EMBED), jnp.float32),
    )(g, wexp, rw3, b_out.reshape(1, _EMBED)).reshape(B, Nq, _EMBED)


def kernel(query, reference_points, value_map, W_off, b_off, W_attn, b_attn,
           W_val, b_val, W_out, b_out):
    B, Nq, C = query.shape
    _, _, H, W = value_map.shape

    value = _value_proj(value_map, W_val, b_val)
    table = value.reshape(B * H * W * _HEADS, C // _HEADS)
    idx, w = _idx_weights(query, reference_points, W_off, b_off,
                          W_attn, b_attn, H, W)
    gathered = _sc_gather(table, idx.reshape(-1))
    return _combine(gathered, w, W_out, b_out, B, Nq)


# speedup vs baseline: 1.3201x; 1.3201x over previous
import functools

import jax
import jax.numpy as jnp
from jax import lax
from jax.experimental import pallas as pl
from jax.experimental.pallas import tpu as pltpu
from jax.experimental.pallas import tpu_sc as plsc

_EMBED = 256
_HEADS = 8
_POINTS = 4
_NCOL = _HEADS * 4 * _POINTS


def _vproj_body(vm_ref, wv_ref, bv_ref, out_ref):
    v = jnp.dot(vm_ref[0], wv_ref[...],
                preferred_element_type=jnp.float32) + bv_ref[...]
    out_ref[0] = v[:, :128]
    out_ref[1] = v[:, 128:]


def _value_proj(value_map, W_val, b_val, T=2000):
    B, C, H, W = value_map.shape
    HW = H * W
    vm = jnp.transpose(value_map.reshape(B, C, HW), (0, 2, 1))
    nt = HW // T
    return pl.pallas_call(
        _vproj_body,
        grid=(B, nt),
        in_specs=[
            pl.BlockSpec((1, T, C), lambda b, t: (b, t, 0)),
            pl.BlockSpec((C, C), lambda b, t: (0, 0)),
            pl.BlockSpec((1, C), lambda b, t: (0, 0)),
        ],
        out_specs=pl.BlockSpec((2, T, 128), lambda b, t, n=nt: (0, b * n + t, 0)),
        out_shape=jax.ShapeDtypeStruct((2, B * HW, 128), jnp.float32),
    )(vm, W_val, b_val.reshape(1, C))


def _idxw_body(q_ref, rpx_ref, rpy_ref, wox_ref, woy_ref, box_ref, boy_ref,
               wa_ref, ba_ref, gsum_ref, idx_ref, w_ref, *, H, W, B):
    Tq = q_ref.shape[1]
    q = q_ref[0]
    offx = jnp.dot(q, wox_ref[...], preferred_element_type=jnp.float32) + box_ref[...]
    offy = jnp.dot(q, woy_ref[...], preferred_element_type=jnp.float32) + boy_ref[...]
    logits = jnp.dot(q, wa_ref[...], preferred_element_type=jnp.float32) + ba_ref[...]
    m = jnp.max(logits, axis=-1, keepdims=True)
    e = jnp.exp(logits - m)
    esum = jnp.dot(e, gsum_ref[...], preferred_element_type=jnp.float32)
    attn = e / esum

    x = (rpx_ref[0] + offx) * W - 0.5
    y = (rpy_ref[0] + offy) * H - 0.5
    x0 = jnp.floor(x)
    fx = x - x0
    y0 = jnp.floor(y)
    fy = y - y0

    col = lax.broadcasted_iota(jnp.int32, (Tq, _NCOL), 1)
    cvec = (col % 16) // 4
    is_x1 = (cvec % 2) == 1
    is_y1 = cvec >= 2

    one = jnp.float32(1.0)
    wx = jnp.where(is_x1, fx, one - fx)
    wy = jnp.where(is_y1, fy, one - fy)
    xf = jnp.where(is_x1, x0 + one, x0)
    yf = jnp.where(is_y1, y0 + one, y0)
    vx = ((xf >= 0.0) & (xf <= W - 1.0)).astype(jnp.float32)
    vy = ((yf >= 0.0) & (yf <= H - 1.0)).astype(jnp.float32)
    xc = jnp.clip(xf, 0.0, W - 1.0).astype(jnp.int32)
    yc = jnp.clip(yf, 0.0, H - 1.0).astype(jnp.int32)

    w_ref[0] = wx * wy * vx * vy * attn

    half = col // 64
    hh = (col // 16) % 4
    base = pl.program_id(0) * (H * W)
    idx_ref[0] = half * (B * H * W * 4) + (base + yc * W + xc) * 4 + hh


def _idx_weights(query, reference_points, W_off, b_off, W_attn, b_attn,
                 H, W, Tq=1000):
    B, Nq, C = query.shape
    Wo = W_off.reshape(C, _HEADS, _POINTS, 2)
    bo = b_off.reshape(_HEADS, _POINTS, 2)
    wox = jnp.broadcast_to(Wo[:, :, None, :, 0], (C, _HEADS, 4, _POINTS)).reshape(C, _NCOL)
    woy = jnp.broadcast_to(Wo[:, :, None, :, 1], (C, _HEADS, 4, _POINTS)).reshape(C, _NCOL)
    box = jnp.broadcast_to(bo[:, None, :, 0], (_HEADS, 4, _POINTS)).reshape(1, _NCOL)
    boy = jnp.broadcast_to(bo[:, None, :, 1], (_HEADS, 4, _POINTS)).reshape(1, _NCOL)
    Wa = W_attn.reshape(C, _HEADS, _POINTS)
    wa = jnp.broadcast_to(Wa[:, :, None, :], (C, _HEADS, 4, _POINTS)).reshape(C, _NCOL)
    ba = jnp.broadcast_to(b_attn.reshape(_HEADS, _POINTS)[:, None, :],
                          (_HEADS, 4, _POINTS)).reshape(1, _NCOL)
    gsum = jnp.kron(jnp.eye(_NCOL // 4, dtype=jnp.float32),
                    jnp.ones((4, 4), jnp.float32))
    rpx = reference_points[:, :, 0:1]
    rpy = reference_points[:, :, 1:2]

    body = functools.partial(_idxw_body, H=H, W=W, B=B)
    return pl.pallas_call(
        body,
        grid=(B, Nq // Tq),
        in_specs=[
            pl.BlockSpec((1, Tq, C), lambda b, t: (b, t, 0)),
            pl.BlockSpec((1, Tq, 1), lambda b, t: (b, t, 0)),
            pl.BlockSpec((1, Tq, 1), lambda b, t: (b, t, 0)),
            pl.BlockSpec((C, _NCOL), lambda b, t: (0, 0)),
            pl.BlockSpec((C, _NCOL), lambda b, t: (0, 0)),
            pl.BlockSpec((1, _NCOL), lambda b, t: (0, 0)),
            pl.BlockSpec((1, _NCOL), lambda b, t: (0, 0)),
            pl.BlockSpec((C, _NCOL), lambda b, t: (0, 0)),
            pl.BlockSpec((1, _NCOL), lambda b, t: (0, 0)),
            pl.BlockSpec((_NCOL, _NCOL), lambda b, t: (0, 0)),
        ],
        out_specs=[
            pl.BlockSpec((1, Tq, _NCOL), lambda b, t: (b, t, 0)),
            pl.BlockSpec((1, Tq, _NCOL), lambda b, t: (b, t, 0)),
        ],
        out_shape=[
            jax.ShapeDtypeStruct((B, Nq, _NCOL), jnp.int32),
            jax.ShapeDtypeStruct((B, Nq, _NCOL), jnp.float32),
        ],
    )(query, rpx, rpy, wox, woy, box, boy, wa, ba, gsum)


_GW = 128


def _sc_gather(table, idx_flat):
    S4 = idx_flat.shape[0]
    D = table.shape[1]
    mesh = plsc.VectorSubcoreMesh(core_axis_name="core",
                                  subcore_axis_name="subcore")

    @functools.partial(
        pl.kernel,
        out_type=jax.ShapeDtypeStruct((S4, D), table.dtype),
        mesh=mesh,
        compiler_params=pltpu.CompilerParams(use_tc_tiling_on_sc=False),
    )
    def gather_kernel(table_hbm, idx_hbm, out_hbm):
        def body(i_vmem, o_vmem):
            pltpu.sync_copy(table_hbm.at[i_vmem.at[0]], o_vmem)

        pltpu.emit_pipeline(
            body,
            grid=(S4 // _GW,),
            in_specs=[pl.BlockSpec((1, _GW), index_map=lambda i: (0, i))],
            out_specs=[pl.BlockSpec((_GW, D), index_map=lambda i: (i, 0))],
            core_axis_name=("core", "subcore"),
            dimension_semantics=(pltpu.PARALLEL,),
        )(idx_hbm, out_hbm)

    return gather_kernel(table, idx_flat.reshape(1, S4))


def _combine_body(g_ref, w_ref, scat_ref, rw_ref, bo_ref, out_ref):
    g = g_ref[0]
    w = w_ref[0]
    wexp = jnp.dot(w.astype(jnp.bfloat16), scat_ref[...],
                   preferred_element_type=jnp.float32)
    gw = (g * wexp).astype(jnp.bfloat16)
    out_ref[0] = jnp.dot(gw, rw_ref[...],
                         preferred_element_type=jnp.float32) + bo_ref[...]


def _combine(gathered, w, W_out, b_out, B, Nq, Tq=400):
    g = gathered.reshape(B, Nq, 128 * 32)
    scat = jnp.kron(jnp.eye(_NCOL, dtype=jnp.bfloat16),
                    jnp.ones((1, 32), jnp.bfloat16))
    rw = jnp.broadcast_to(
        W_out.reshape(_HEADS, 1, 32, _EMBED),
        (_HEADS, 16, 32, _EMBED)).reshape(4096, _EMBED).astype(jnp.bfloat16)
    return pl.pallas_call(
        _combine_body,
        grid=(B, Nq // Tq),
        in_specs=[
            pl.BlockSpec((1, Tq, 4096), lambda b, t: (b, t, 0)),
            pl.BlockSpec((1, Tq, _NCOL), lambda b, t: (b, t, 0)),
            pl.BlockSpec((_NCOL, 4096), lambda b, t: (0, 0)),
            pl.BlockSpec((4096, _EMBED), lambda b, t: (0, 0)),
            pl.BlockSpec((1, _EMBED), lambda b, t: (0, 0)),
        ],
        out_specs=pl.BlockSpec((1, Tq, _EMBED), lambda b, t: (b, t, 0)),
        out_shape=jax.ShapeDtypeStruct((B, Nq, _EMBED), jnp.float32),
    )(g, w, scat, rw, b_out.reshape(1, _EMBED))


def kernel(query, reference_points, value_map, W_off, b_off, W_attn, b_attn,
           W_val, b_val, W_out, b_out):
    B, Nq, C = query.shape
    _, _, H, W = value_map.shape

    value = _value_proj(value_map, W_val, b_val)
    table = value.reshape(B * H * W * _HEADS, C // _HEADS)
    idx, w = _idx_weights(query, reference_points, W_off, b_off,
                          W_attn, b_attn, H, W)
    gathered = _sc_gather(table, idx.reshape(-1))
    return _combine(gathered, w, W_out, b_out, B, Nq)


# speedup vs baseline: 1.4131x; 1.0705x over previous
import functools

import jax
import jax.numpy as jnp
from jax import lax
from jax.experimental import pallas as pl
from jax.experimental.pallas import tpu as pltpu
from jax.experimental.pallas import tpu_sc as plsc

_EMBED = 256
_HEADS = 8
_POINTS = 4
_NCOL = _HEADS * 4 * _POINTS


def _vproj_body(vm_ref, wv_ref, bv_ref, out_ref):
    v = jnp.dot(vm_ref[0], wv_ref[...],
                preferred_element_type=jnp.float32) + bv_ref[...]
    out_ref[0] = v[:, :128]
    out_ref[1] = v[:, 128:]


def _value_proj(value_map, W_val, b_val, T=2000):
    B, C, H, W = value_map.shape
    HW = H * W
    vm = jnp.transpose(value_map.reshape(B, C, HW), (0, 2, 1))
    nt = HW // T
    return pl.pallas_call(
        _vproj_body,
        grid=(B, nt),
        in_specs=[
            pl.BlockSpec((1, T, C), lambda b, t: (b, t, 0)),
            pl.BlockSpec((C, C), lambda b, t: (0, 0)),
            pl.BlockSpec((1, C), lambda b, t: (0, 0)),
        ],
        out_specs=pl.BlockSpec((2, T, 128), lambda b, t, n=nt: (0, b * n + t, 0)),
        out_shape=jax.ShapeDtypeStruct((2, B * HW, 128), jnp.float32),
    )(vm, W_val, b_val.reshape(1, C))


def _idxw_body(q_ref, rpx_ref, rpy_ref, wox_ref, woy_ref, box_ref, boy_ref,
               wa_ref, ba_ref, gsum_ref, idx_ref, w_ref, *, H, W, B):
    Tq = q_ref.shape[1]
    q = q_ref[0]
    offx = jnp.dot(q, wox_ref[...], preferred_element_type=jnp.float32) + box_ref[...]
    offy = jnp.dot(q, woy_ref[...], preferred_element_type=jnp.float32) + boy_ref[...]
    logits = jnp.dot(q, wa_ref[...], preferred_element_type=jnp.float32) + ba_ref[...]
    m = jnp.max(logits, axis=-1, keepdims=True)
    e = jnp.exp(logits - m)
    esum = jnp.dot(e, gsum_ref[...], preferred_element_type=jnp.float32)
    attn = e / esum

    x = (rpx_ref[0] + offx) * W - 0.5
    y = (rpy_ref[0] + offy) * H - 0.5
    x0 = jnp.floor(x)
    fx = x - x0
    y0 = jnp.floor(y)
    fy = y - y0

    col = lax.broadcasted_iota(jnp.int32, (Tq, _NCOL), 1)
    cvec = (col % 16) // 4
    is_x1 = (cvec % 2) == 1
    is_y1 = cvec >= 2

    one = jnp.float32(1.0)
    wx = jnp.where(is_x1, fx, one - fx)
    wy = jnp.where(is_y1, fy, one - fy)
    xf = jnp.where(is_x1, x0 + one, x0)
    yf = jnp.where(is_y1, y0 + one, y0)
    vx = ((xf >= 0.0) & (xf <= W - 1.0)).astype(jnp.float32)
    vy = ((yf >= 0.0) & (yf <= H - 1.0)).astype(jnp.float32)
    xc = jnp.clip(xf, 0.0, W - 1.0).astype(jnp.int32)
    yc = jnp.clip(yf, 0.0, H - 1.0).astype(jnp.int32)

    w_ref[0] = wx * wy * vx * vy * attn

    half = col // 64
    hh = (col // 16) % 4
    base = pl.program_id(0) * (H * W)
    idx_ref[0] = half * (B * H * W * 4) + (base + yc * W + xc) * 4 + hh


def _idx_weights(query, reference_points, W_off, b_off, W_attn, b_attn,
                 H, W, Tq=1000):
    B, Nq, C = query.shape
    Wo = W_off.reshape(C, _HEADS, _POINTS, 2)
    bo = b_off.reshape(_HEADS, _POINTS, 2)
    wox = jnp.broadcast_to(Wo[:, :, None, :, 0], (C, _HEADS, 4, _POINTS)).reshape(C, _NCOL)
    woy = jnp.broadcast_to(Wo[:, :, None, :, 1], (C, _HEADS, 4, _POINTS)).reshape(C, _NCOL)
    box = jnp.broadcast_to(bo[:, None, :, 0], (_HEADS, 4, _POINTS)).reshape(1, _NCOL)
    boy = jnp.broadcast_to(bo[:, None, :, 1], (_HEADS, 4, _POINTS)).reshape(1, _NCOL)
    Wa = W_attn.reshape(C, _HEADS, _POINTS)
    wa = jnp.broadcast_to(Wa[:, :, None, :], (C, _HEADS, 4, _POINTS)).reshape(C, _NCOL)
    ba = jnp.broadcast_to(b_attn.reshape(_HEADS, _POINTS)[:, None, :],
                          (_HEADS, 4, _POINTS)).reshape(1, _NCOL)
    gsum = jnp.kron(jnp.eye(_NCOL // 4, dtype=jnp.float32),
                    jnp.ones((4, 4), jnp.float32))
    rpx = reference_points[:, :, 0:1]
    rpy = reference_points[:, :, 1:2]

    body = functools.partial(_idxw_body, H=H, W=W, B=B)
    return pl.pallas_call(
        body,
        grid=(B, Nq // Tq),
        in_specs=[
            pl.BlockSpec((1, Tq, C), lambda b, t: (b, t, 0)),
            pl.BlockSpec((1, Tq, 1), lambda b, t: (b, t, 0)),
            pl.BlockSpec((1, Tq, 1), lambda b, t: (b, t, 0)),
            pl.BlockSpec((C, _NCOL), lambda b, t: (0, 0)),
            pl.BlockSpec((C, _NCOL), lambda b, t: (0, 0)),
            pl.BlockSpec((1, _NCOL), lambda b, t: (0, 0)),
            pl.BlockSpec((1, _NCOL), lambda b, t: (0, 0)),
            pl.BlockSpec((C, _NCOL), lambda b, t: (0, 0)),
            pl.BlockSpec((1, _NCOL), lambda b, t: (0, 0)),
            pl.BlockSpec((_NCOL, _NCOL), lambda b, t: (0, 0)),
        ],
        out_specs=[
            pl.BlockSpec((1, Tq, _NCOL), lambda b, t: (b, t, 0)),
            pl.BlockSpec((1, Tq, _NCOL), lambda b, t: (b, t, 0)),
        ],
        out_shape=[
            jax.ShapeDtypeStruct((B, Nq, _NCOL), jnp.int32),
            jax.ShapeDtypeStruct((B, Nq, _NCOL), jnp.float32),
        ],
    )(query, rpx, rpy, wox, woy, box, boy, wa, ba, gsum)


_GW = 128


def _sc_gather(table, idx_flat):
    S4 = idx_flat.shape[0]
    D = table.shape[1]
    mesh = plsc.VectorSubcoreMesh(core_axis_name="core",
                                  subcore_axis_name="subcore")

    @functools.partial(
        pl.kernel,
        out_type=jax.ShapeDtypeStruct((S4, D), table.dtype),
        mesh=mesh,
        compiler_params=pltpu.CompilerParams(use_tc_tiling_on_sc=False),
    )
    def gather_kernel(table_hbm, idx_hbm, out_hbm):
        def body(i_vmem, o_vmem):
            pltpu.sync_copy(table_hbm.at[i_vmem.at[0]], o_vmem)

        pltpu.emit_pipeline(
            body,
            grid=(S4 // _GW,),
            in_specs=[pl.BlockSpec((1, _GW), index_map=lambda i: (0, i))],
            out_specs=[pl.BlockSpec((_GW, D), index_map=lambda i: (i, 0))],
            core_axis_name=("core", "subcore"),
            dimension_semantics=(pltpu.PARALLEL,),
        )(idx_hbm, out_hbm)

    return gather_kernel(table, idx_flat.reshape(1, S4))


def _combine_body(g_ref, w_ref, scat_ref, rw_ref, bo_ref, out_ref):
    g = g_ref[0]
    w = w_ref[0]
    wexp = jnp.dot(w.astype(jnp.bfloat16), scat_ref[...],
                   preferred_element_type=jnp.float32)
    gw = (g * wexp).astype(jnp.bfloat16)
    out_ref[0] = jnp.dot(gw, rw_ref[...],
                         preferred_element_type=jnp.float32) + bo_ref[...]


def _combine(gathered, w, W_out, b_out, B, Nq, Tq=400):
    g = gathered.reshape(B, Nq, 128 * 32)
    scat = jnp.kron(jnp.eye(_NCOL, dtype=jnp.bfloat16),
                    jnp.ones((1, 32), jnp.bfloat16))
    rw = jnp.broadcast_to(
        W_out.reshape(_HEADS, 1, 32, _EMBED),
        (_HEADS, 16, 32, _EMBED)).reshape(4096, _EMBED).astype(jnp.bfloat16)
    return pl.pallas_call(
        _combine_body,
        grid=(B, Nq // Tq),
        in_specs=[
            pl.BlockSpec((1, Tq, 4096), lambda b, t: (b, t, 0)),
            pl.BlockSpec((1, Tq, _NCOL), lambda b, t: (b, t, 0)),
            pl.BlockSpec((_NCOL, 4096), lambda b, t: (0, 0)),
            pl.BlockSpec((4096, _EMBED), lambda b, t: (0, 0)),
            pl.BlockSpec((1, _EMBED), lambda b, t: (0, 0)),
        ],
        out_specs=pl.BlockSpec((1, Tq, _EMBED), lambda b, t: (b, t, 0)),
        out_shape=jax.ShapeDtypeStruct((B, Nq, _EMBED), jnp.float32),
    )(g, w, scat, rw, b_out.reshape(1, _EMBED))


def kernel(query, reference_points, value_map, W_off, b_off, W_attn, b_attn,
           W_val, b_val, W_out, b_out):
    B, Nq, C = query.shape
    _, _, H, W = value_map.shape

    value = _value_proj(value_map, W_val, b_val)
    table = value.reshape(B * H * W * _HEADS, C // _HEADS)
    idx, w = _idx_weights(query, reference_points, W_off, b_off,
                          W_attn, b_attn, H, W)
    nchunk = 5
    qc = Nq // nchunk
    outs = []
    for c in range(nchunk):
        idx_c = idx[:, c * qc:(c + 1) * qc]
        w_c = w[:, c * qc:(c + 1) * qc]
        gathered = _sc_gather(table, idx_c.reshape(-1))
        outs.append(_combine(gathered, w_c, W_out, b_out, B, qc))
    return jnp.concatenate(outs, axis=1)
